# Initial kernel scaffold; baseline (speedup 1.0000x reference)
import jax
import jax.numpy as jnp
from jax.experimental import pallas as pl


def kernel(x, edge_index, W1, W2):
    raise NotImplementedError("write your pallas kernel here")


# speedup vs baseline: 31.2670x; 31.2670x over previous
import functools

import jax
import jax.numpy as jnp
from jax import lax
from jax.experimental import pallas as pl
from jax.experimental.pallas import tpu as pltpu
from jax.experimental.pallas import tpu_sc as plsc

N = 10000
E = 320000
D_IN = 128
D = 4
DP = 8

NC, NS = 2, 16
NW = NC * NS
CHUNK = 128
CPW = -(-E // (NW * CHUNK))
E_PAD = NW * CPW * CHUNK
N_PAD = 10112
ROWS_PER_TILE = N_PAD // NS

_mesh = plsc.VectorSubcoreMesh(core_axis_name="c", subcore_axis_name="s")
_sc_params = pltpu.CompilerParams(use_tc_tiling_on_sc=False)


@functools.partial(
    pl.kernel,
    out_type=jax.ShapeDtypeStruct((NC, N_PAD, DP), jnp.float32),
    mesh=_mesh,
    scratch_types=[
        pltpu.VMEM((CPW, CHUNK), jnp.int32),
        pltpu.VMEM((CHUNK, DP), jnp.float32),
        pltpu.VMEM_SHARED((N_PAD, DP), jnp.float32),
    ],
    compiler_params=_sc_params,
)
def _deg_sc(dst3, ones_hbm, zer_hbm, deg_out, didx, ones_v, acc_sh):
    c = lax.axis_index("c")
    s = lax.axis_index("s")
    wid = c * NS + s
    pltpu.sync_copy(dst3.at[wid], didx)
    pltpu.sync_copy(ones_hbm, ones_v)

    @pl.when(s == 0)
    def _():
        pltpu.sync_copy(zer_hbm, acc_sh)

    plsc.subcore_barrier()

    def body(j, carry):
        pltpu.sync_copy(ones_v, acc_sh.at[didx.at[j]], add=True)
        return carry

    lax.fori_loop(0, CPW, body, 0)
    plsc.subcore_barrier()
    pltpu.sync_copy(
        acc_sh.at[pl.ds(s * ROWS_PER_TILE, ROWS_PER_TILE)],
        deg_out.at[c, pl.ds(s * ROWS_PER_TILE, ROWS_PER_TILE)],
    )


@functools.partial(
    pl.kernel,
    out_type=jax.ShapeDtypeStruct((NC, N_PAD, DP), jnp.float32),
    mesh=_mesh,
    scratch_types=[
        pltpu.VMEM((CPW, CHUNK), jnp.int32),
        pltpu.VMEM((CPW, CHUNK), jnp.int32),
        pltpu.VMEM((CHUNK, DP), jnp.float32),
        pltpu.VMEM_SHARED((N_PAD, DP), jnp.float32),
        pltpu.SemaphoreType.DMA,
    ],
    compiler_params=_sc_params,
)
def _msg_sc(src3, dst3, y_hbm, zer_hbm, z_out, sidx, didx, msgs, acc_sh, sem):
    c = lax.axis_index("c")
    s = lax.axis_index("s")
    wid = c * NS + s
    pltpu.sync_copy(src3.at[wid], sidx)
    pltpu.sync_copy(dst3.at[wid], didx)

    @pl.when(s == 0)
    def _():
        pltpu.sync_copy(zer_hbm, acc_sh)

    plsc.subcore_barrier()

    def body(j, carry):
        pltpu.async_copy(y_hbm.at[sidx.at[j]], msgs, sem).wait()
        pltpu.sync_copy(msgs, acc_sh.at[didx.at[j]], add=True)
        return carry

    lax.fori_loop(0, CPW, body, 0)
    plsc.subcore_barrier()
    pltpu.sync_copy(
        acc_sh.at[pl.ds(s * ROWS_PER_TILE, ROWS_PER_TILE)],
        z_out.at[c, pl.ds(s * ROWS_PER_TILE, ROWS_PER_TILE)],
    )


def _tc1_body(x_ref, w_ref, degp_ref, y_ref, dinv_ref):
    deg = degp_ref[0] + degp_ref[1] + 1.0
    dinv = lax.rsqrt(jnp.maximum(deg, 1.0))
    h = jnp.dot(x_ref[...], w_ref[...], preferred_element_type=jnp.float32)
    pad = jnp.zeros((N, DP - D), jnp.float32)
    y_ref[...] = jnp.concatenate([h * dinv, pad], axis=1)
    dinv_ref[...] = dinv


def _tc2_body(zp_ref, y1_ref, dinv_ref, w2_ref, h_ref, y2_ref):
    z8 = zp_ref[0] + zp_ref[1] + y1_ref[...]
    g8 = jnp.tanh(z8 * dinv_ref[...])
    g = g8[:, :D]
    h_ref[...] = g
    pad = jnp.zeros((N, DP - D), jnp.float32)
    y2 = jnp.dot(g, w2_ref[...], preferred_element_type=jnp.float32) * dinv_ref[...]
    y2_ref[...] = jnp.concatenate([y2, pad], axis=1)


def _tc3_body(zp_ref, y2_ref, dinv_ref, out_ref):
    z8 = zp_ref[0] + zp_ref[1] + y2_ref[...]
    out_ref[...] = z8[:, :D] * dinv_ref[...]


def kernel(x, edge_index, W1, W2):
    src = edge_index[0].astype(jnp.int32)
    dst = edge_index[1].astype(jnp.int32)
    pad = E_PAD - E
    src3 = jnp.concatenate([src, jnp.zeros((pad,), jnp.int32)]).reshape(
        NW, CPW, CHUNK
    )
    dst3 = jnp.concatenate([dst, jnp.full((pad,), N, jnp.int32)]).reshape(
        NW, CPW, CHUNK
    )
    ones8 = jnp.ones((CHUNK, DP), jnp.float32)
    zer8 = jnp.zeros((N_PAD, DP), jnp.float32)

    degp = _deg_sc(dst3, ones8, zer8)
    deg2 = degp[:, :N, 0:1]

    y1, dinv = pl.pallas_call(
        _tc1_body,
        out_shape=[
            jax.ShapeDtypeStruct((N, DP), jnp.float32),
            jax.ShapeDtypeStruct((N, 1), jnp.float32),
        ],
    )(x, W1, deg2)

    z1p = _msg_sc(src3, dst3, y1, zer8)

    h, y2 = pl.pallas_call(
        _tc2_body,
        out_shape=[
            jax.ShapeDtypeStruct((N, D), jnp.float32),
            jax.ShapeDtypeStruct((N, DP), jnp.float32),
        ],
    )(z1p[:, :N], y1, dinv, W2)

    z2p = _msg_sc(src3, dst3, y2, zer8)

    out = pl.pallas_call(
        _tc3_body,
        out_shape=jax.ShapeDtypeStruct((N, D), jnp.float32),
    )(z2p[:, :N], y2, dinv)

    return (out, h)


# speedup vs baseline: 38.6439x; 1.2359x over previous
import functools

import jax
import jax.numpy as jnp
from jax import lax
from jax.experimental import pallas as pl
from jax.experimental.pallas import tpu as pltpu
from jax.experimental.pallas import tpu_sc as plsc

N = 10000
E = 320000
D_IN = 128
D = 4
DP = 8

NC, NS = 2, 16
NW = NC * NS
CHUNK = 128
NBUF = 4
CPW = 80
E_PAD = NW * CPW * CHUNK
N_PAD = 10112
ROWS_PER_TILE = N_PAD // NS

_mesh = plsc.VectorSubcoreMesh(core_axis_name="c", subcore_axis_name="s")
_sc_params = pltpu.CompilerParams(use_tc_tiling_on_sc=False)


@functools.partial(
    pl.kernel,
    out_type=jax.ShapeDtypeStruct((NC, N_PAD, DP), jnp.float32),
    mesh=_mesh,
    scratch_types=[
        pltpu.VMEM((CPW, CHUNK), jnp.int32),
        pltpu.VMEM((CHUNK, DP), jnp.float32),
        pltpu.VMEM_SHARED((N_PAD, DP), jnp.float32),
    ],
    compiler_params=_sc_params,
)
def _deg_sc(dst3, ones_hbm, zer_hbm, deg_out, didx, ones_v, acc_sh):
    c = lax.axis_index("c")
    s = lax.axis_index("s")
    wid = c * NS + s
    pltpu.sync_copy(dst3.at[wid], didx)
    pltpu.sync_copy(ones_hbm, ones_v)

    @pl.when(s == 0)
    def _():
        pltpu.sync_copy(zer_hbm, acc_sh)

    plsc.subcore_barrier()

    def body(j, carry):
        pltpu.sync_copy(ones_v, acc_sh.at[didx.at[j]], add=True)
        return carry

    lax.fori_loop(0, CPW, body, 0)
    plsc.subcore_barrier()
    pltpu.sync_copy(
        acc_sh.at[pl.ds(s * ROWS_PER_TILE, ROWS_PER_TILE)],
        deg_out.at[c, pl.ds(s * ROWS_PER_TILE, ROWS_PER_TILE)],
    )


@functools.partial(
    pl.kernel,
    out_type=jax.ShapeDtypeStruct((NC, N_PAD, DP), jnp.float32),
    mesh=_mesh,
    scratch_types=[
        pltpu.VMEM((CPW, CHUNK), jnp.int32),
        pltpu.VMEM((CPW, CHUNK), jnp.int32),
        pltpu.VMEM((NBUF, CHUNK, DP), jnp.float32),
        pltpu.VMEM_SHARED((N_PAD, DP), jnp.float32),
        pltpu.SemaphoreType.DMA((NBUF,)),
    ],
    compiler_params=_sc_params,
)
def _msg_sc(src3, dst3, y_hbm, zer_hbm, z_out, sidx, didx, msgs, acc_sh, sems):
    c = lax.axis_index("c")
    s = lax.axis_index("s")
    wid = c * NS + s
    pltpu.sync_copy(src3.at[wid], sidx)
    pltpu.sync_copy(dst3.at[wid], didx)

    @pl.when(s == 0)
    def _():
        pltpu.sync_copy(zer_hbm, acc_sh)

    plsc.subcore_barrier()

    for b in range(NBUF):
        pltpu.async_copy(y_hbm.at[sidx.at[b]], msgs.at[b], sems.at[b])

    def body(jo, carry):
        for b in range(NBUF):
            j = jo * NBUF + b
            pltpu.make_async_copy(y_hbm.at[sidx.at[j]], msgs.at[b],
                                  sems.at[b]).wait()
            pltpu.sync_copy(msgs.at[b], acc_sh.at[didx.at[j]], add=True)
            jn = j + NBUF

            @pl.when(jn < CPW)
            def _():
                pltpu.async_copy(y_hbm.at[sidx.at[jn]], msgs.at[b],
                                 sems.at[b])
        return carry

    lax.fori_loop(0, CPW // NBUF, body, 0)
    plsc.subcore_barrier()
    pltpu.sync_copy(
        acc_sh.at[pl.ds(s * ROWS_PER_TILE, ROWS_PER_TILE)],
        z_out.at[c, pl.ds(s * ROWS_PER_TILE, ROWS_PER_TILE)],
    )


def _tc1_body(x_ref, w_ref, degp_ref, y_ref, dinv_ref):
    deg = degp_ref[0] + degp_ref[1] + 1.0
    dinv = lax.rsqrt(jnp.maximum(deg, 1.0))
    h = jnp.dot(x_ref[...], w_ref[...], preferred_element_type=jnp.float32)
    pad = jnp.zeros((N, DP - D), jnp.float32)
    y_ref[...] = jnp.concatenate([h * dinv, pad], axis=1)
    dinv_ref[...] = dinv


def _tc2_body(zp_ref, y1_ref, dinv_ref, w2_ref, h_ref, y2_ref):
    z8 = zp_ref[0] + zp_ref[1] + y1_ref[...]
    g8 = jnp.tanh(z8 * dinv_ref[...])
    g = g8[:, :D]
    h_ref[...] = g
    pad = jnp.zeros((N, DP - D), jnp.float32)
    y2 = jnp.dot(g, w2_ref[...], preferred_element_type=jnp.float32) * dinv_ref[...]
    y2_ref[...] = jnp.concatenate([y2, pad], axis=1)


def _tc3_body(zp_ref, y2_ref, dinv_ref, out_ref):
    z8 = zp_ref[0] + zp_ref[1] + y2_ref[...]
    out_ref[...] = z8[:, :D] * dinv_ref[...]


def kernel(x, edge_index, W1, W2):
    src = edge_index[0].astype(jnp.int32)
    dst = edge_index[1].astype(jnp.int32)
    pad = E_PAD - E
    src3 = jnp.concatenate([src, jnp.zeros((pad,), jnp.int32)]).reshape(
        NW, CPW, CHUNK
    )
    dst3 = jnp.concatenate([dst, jnp.full((pad,), N, jnp.int32)]).reshape(
        NW, CPW, CHUNK
    )
    ones8 = jnp.ones((CHUNK, DP), jnp.float32)
    zer8 = jnp.zeros((N_PAD, DP), jnp.float32)

    degp = _deg_sc(dst3, ones8, zer8)
    deg2 = degp[:, :N, 0:1]

    y1, dinv = pl.pallas_call(
        _tc1_body,
        out_shape=[
            jax.ShapeDtypeStruct((N, DP), jnp.float32),
            jax.ShapeDtypeStruct((N, 1), jnp.float32),
        ],
    )(x, W1, deg2)

    z1p = _msg_sc(src3, dst3, y1, zer8)

    h, y2 = pl.pallas_call(
        _tc2_body,
        out_shape=[
            jax.ShapeDtypeStruct((N, D), jnp.float32),
            jax.ShapeDtypeStruct((N, DP), jnp.float32),
        ],
    )(z1p[:, :N], y1, dinv, W2)

    z2p = _msg_sc(src3, dst3, y2, zer8)

    out = pl.pallas_call(
        _tc3_body,
        out_shape=jax.ShapeDtypeStruct((N, D), jnp.float32),
    )(z2p[:, :N], y2, dinv)

    return (out, h)


# speedup vs baseline: 53.2920x; 1.3791x over previous
import functools

import jax
import jax.numpy as jnp
from jax import lax
from jax.experimental import pallas as pl
from jax.experimental.pallas import tpu as pltpu
from jax.experimental.pallas import tpu_sc as plsc

N = 10000
E = 320000
D_IN = 128
D = 4
DP = 8

NC, NS = 2, 16
NW = NC * NS
CHUNK = 128
NBUF = 4
CPW = 80
E_PAD = NW * CPW * CHUNK
N_PAD = 10112
ROWS_PER_TILE = N_PAD // NS

_mesh = plsc.VectorSubcoreMesh(core_axis_name="c", subcore_axis_name="s")
_sc_params = pltpu.CompilerParams(use_tc_tiling_on_sc=False)


@functools.partial(
    pl.kernel,
    out_type=jax.ShapeDtypeStruct((NC, N_PAD, DP), jnp.float32),
    mesh=_mesh,
    scratch_types=[
        pltpu.VMEM((CPW, CHUNK), jnp.int32),
        pltpu.VMEM((CHUNK, DP), jnp.float32),
        pltpu.VMEM_SHARED((N_PAD, DP), jnp.float32),
    ],
    compiler_params=_sc_params,
)
def _deg_sc(dst3, ones_hbm, zer_hbm, deg_out, didx, ones_v, acc_sh):
    c = lax.axis_index("c")
    s = lax.axis_index("s")
    wid = c * NS + s
    pltpu.sync_copy(dst3.at[wid], didx)
    pltpu.sync_copy(ones_hbm, ones_v)

    @pl.when(s == 0)
    def _():
        pltpu.sync_copy(zer_hbm, acc_sh)

    plsc.subcore_barrier()

    def body(j, carry):
        pltpu.sync_copy(ones_v, acc_sh.at[didx.at[j]], add=True)
        return carry

    lax.fori_loop(0, CPW, body, 0)
    plsc.subcore_barrier()
    pltpu.sync_copy(
        acc_sh.at[pl.ds(s * ROWS_PER_TILE, ROWS_PER_TILE)],
        deg_out.at[c, pl.ds(s * ROWS_PER_TILE, ROWS_PER_TILE)],
    )


@functools.partial(
    pl.kernel,
    out_type=jax.ShapeDtypeStruct((NC, N_PAD, DP), jnp.float32),
    mesh=_mesh,
    scratch_types=[
        pltpu.VMEM((CPW, CHUNK), jnp.int32),
        pltpu.VMEM((CPW, CHUNK), jnp.int32),
        pltpu.VMEM((NBUF, CHUNK, DP), jnp.float32),
        pltpu.VMEM_SHARED((N_PAD, DP), jnp.float32),
        pltpu.VMEM_SHARED((N_PAD, DP), jnp.float32),
        pltpu.SemaphoreType.DMA((NBUF,)),
    ],
    compiler_params=_sc_params,
)
def _msg_sc(src3, dst3, y_hbm, zer_hbm, z_out, sidx, didx, msgs, acc_sh,
            y_sh, sems):
    c = lax.axis_index("c")
    s = lax.axis_index("s")
    wid = c * NS + s
    pltpu.sync_copy(src3.at[wid], sidx)
    pltpu.sync_copy(dst3.at[wid], didx)

    pltpu.sync_copy(
        y_hbm.at[pl.ds(s * ROWS_PER_TILE, ROWS_PER_TILE)],
        y_sh.at[pl.ds(s * ROWS_PER_TILE, ROWS_PER_TILE)],
    )

    @pl.when(s == 0)
    def _():
        pltpu.sync_copy(zer_hbm, acc_sh)

    plsc.subcore_barrier()

    for b in range(NBUF):
        pltpu.async_copy(y_sh.at[sidx.at[b]], msgs.at[b], sems.at[b])

    def body(jo, carry):
        for b in range(NBUF):
            j = jo * NBUF + b
            pltpu.make_async_copy(y_sh.at[sidx.at[j]], msgs.at[b],
                                  sems.at[b]).wait()
            pltpu.sync_copy(msgs.at[b], acc_sh.at[didx.at[j]], add=True)
            jn = j + NBUF

            @pl.when(jn < CPW)
            def _():
                pltpu.async_copy(y_sh.at[sidx.at[jn]], msgs.at[b],
                                 sems.at[b])
        return carry

    lax.fori_loop(0, CPW // NBUF, body, 0)
    plsc.subcore_barrier()
    pltpu.sync_copy(
        acc_sh.at[pl.ds(s * ROWS_PER_TILE, ROWS_PER_TILE)],
        z_out.at[c, pl.ds(s * ROWS_PER_TILE, ROWS_PER_TILE)],
    )


def _tc1_body(x_ref, w_ref, degp_ref, y_ref, dinv_ref):
    deg = degp_ref[0] + degp_ref[1] + 1.0
    dinv = lax.rsqrt(jnp.maximum(deg, 1.0))
    h = jnp.dot(x_ref[...], w_ref[...], preferred_element_type=jnp.float32)
    pad = jnp.zeros((N, DP - D), jnp.float32)
    y_ref[...] = jnp.concatenate([h * dinv, pad], axis=1)
    dinv_ref[...] = dinv


def _tc2_body(zp_ref, y1_ref, dinv_ref, w2_ref, h_ref, y2_ref):
    z8 = zp_ref[0] + zp_ref[1] + y1_ref[...]
    g8 = jnp.tanh(z8 * dinv_ref[...])
    g = g8[:, :D]
    h_ref[...] = g
    pad = jnp.zeros((N, DP - D), jnp.float32)
    y2 = jnp.dot(g, w2_ref[...], preferred_element_type=jnp.float32) * dinv_ref[...]
    y2_ref[...] = jnp.concatenate([y2, pad], axis=1)


def _tc3_body(zp_ref, y2_ref, dinv_ref, out_ref):
    z8 = zp_ref[0] + zp_ref[1] + y2_ref[...]
    out_ref[...] = z8[:, :D] * dinv_ref[...]


def kernel(x, edge_index, W1, W2):
    src = edge_index[0].astype(jnp.int32)
    dst = edge_index[1].astype(jnp.int32)
    pad = E_PAD - E
    src3 = jnp.concatenate([src, jnp.zeros((pad,), jnp.int32)]).reshape(
        NW, CPW, CHUNK
    )
    dst3 = jnp.concatenate([dst, jnp.full((pad,), N, jnp.int32)]).reshape(
        NW, CPW, CHUNK
    )
    ones8 = jnp.ones((CHUNK, DP), jnp.float32)
    zer8 = jnp.zeros((N_PAD, DP), jnp.float32)

    degp = _deg_sc(dst3, ones8, zer8)
    deg2 = degp[:, :N, 0:1]

    y1, dinv = pl.pallas_call(
        _tc1_body,
        out_shape=[
            jax.ShapeDtypeStruct((N, DP), jnp.float32),
            jax.ShapeDtypeStruct((N, 1), jnp.float32),
        ],
    )(x, W1, deg2)

    z1p = _msg_sc(src3, dst3, y1, zer8)

    h, y2 = pl.pallas_call(
        _tc2_body,
        out_shape=[
            jax.ShapeDtypeStruct((N, D), jnp.float32),
            jax.ShapeDtypeStruct((N, DP), jnp.float32),
        ],
    )(z1p[:, :N], y1, dinv, W2)

    z2p = _msg_sc(src3, dst3, y2, zer8)

    out = pl.pallas_call(
        _tc3_body,
        out_shape=jax.ShapeDtypeStruct((N, D), jnp.float32),
    )(z2p[:, :N], y2, dinv)

    return (out, h)


# speedup vs baseline: 53.3389x; 1.0009x over previous
import functools

import jax
import jax.numpy as jnp
from jax import lax
from jax.experimental import pallas as pl
from jax.experimental.pallas import tpu as pltpu
from jax.experimental.pallas import tpu_sc as plsc

N = 10000
E = 320000
D_IN = 128
D = 4
DP = 8

NC, NS = 2, 16
NW = NC * NS
CHUNK = 128
NBUF = 4
CPW = 80
E_PAD = NW * CPW * CHUNK
N_PAD = 10112
ROWS_PER_TILE = N_PAD // NS

_mesh = plsc.VectorSubcoreMesh(core_axis_name="c", subcore_axis_name="s")
_sc_params = pltpu.CompilerParams(use_tc_tiling_on_sc=False)


@functools.partial(
    pl.kernel,
    out_type=jax.ShapeDtypeStruct((NC, N_PAD, DP), jnp.float32),
    mesh=_mesh,
    scratch_types=[
        pltpu.VMEM((CPW, CHUNK), jnp.int32),
        pltpu.VMEM((CHUNK, DP), jnp.float32),
        pltpu.VMEM_SHARED((N_PAD, DP), jnp.float32),
    ],
    compiler_params=_sc_params,
)
def _deg_sc(dst3, ones_hbm, zer_hbm, deg_out, didx, ones_v, acc_sh):
    c = lax.axis_index("c")
    s = lax.axis_index("s")
    wid = c * NS + s
    pltpu.sync_copy(dst3.at[wid], didx)
    pltpu.sync_copy(ones_hbm, ones_v)

    @pl.when(s == 0)
    def _():
        pltpu.sync_copy(zer_hbm, acc_sh)

    plsc.subcore_barrier()

    def body(j, carry):
        pltpu.sync_copy(ones_v, acc_sh.at[didx.at[j]], add=True)
        return carry

    lax.fori_loop(0, CPW, body, 0)
    plsc.subcore_barrier()
    pltpu.sync_copy(
        acc_sh.at[pl.ds(s * ROWS_PER_TILE, ROWS_PER_TILE)],
        deg_out.at[c, pl.ds(s * ROWS_PER_TILE, ROWS_PER_TILE)],
    )


@functools.partial(
    pl.kernel,
    out_type=jax.ShapeDtypeStruct((NC, N_PAD, DP), jnp.float32),
    mesh=_mesh,
    scratch_types=[
        pltpu.VMEM((CPW, CHUNK), jnp.int32),
        pltpu.VMEM((CPW, CHUNK), jnp.int32),
        pltpu.VMEM((NBUF, CHUNK, DP), jnp.float32),
        pltpu.VMEM_SHARED((N_PAD, DP), jnp.float32),
        pltpu.VMEM_SHARED((N_PAD, DP), jnp.float32),
        pltpu.SemaphoreType.DMA((NBUF,)),
    ],
    compiler_params=_sc_params,
)
def _msg_sc(src3, dst3, y_hbm, zer_hbm, z_out, sidx, didx, msgs, acc_sh,
            y_sh, sems):
    c = lax.axis_index("c")
    s = lax.axis_index("s")
    wid = c * NS + s
    pltpu.sync_copy(src3.at[wid], sidx)
    pltpu.sync_copy(dst3.at[wid], didx)

    pltpu.sync_copy(
        y_hbm.at[pl.ds(s * ROWS_PER_TILE, ROWS_PER_TILE)],
        y_sh.at[pl.ds(s * ROWS_PER_TILE, ROWS_PER_TILE)],
    )

    @pl.when(s == 0)
    def _():
        pltpu.sync_copy(zer_hbm, acc_sh)

    plsc.subcore_barrier()

    for b in range(NBUF):
        pltpu.async_copy(y_sh.at[sidx.at[b]], msgs.at[b], sems.at[b])

    def body(jo, carry):
        for b in range(NBUF):
            j = jo * NBUF + b
            pltpu.make_async_copy(y_sh.at[sidx.at[j]], msgs.at[b],
                                  sems.at[b]).wait()
            pltpu.sync_copy(msgs.at[b], acc_sh.at[didx.at[j]], add=True)
            jn = j + NBUF

            @pl.when(jn < CPW)
            def _():
                pltpu.async_copy(y_sh.at[sidx.at[jn]], msgs.at[b],
                                 sems.at[b])
        return carry

    lax.fori_loop(0, CPW // NBUF, body, 0)
    plsc.subcore_barrier()
    pltpu.sync_copy(
        acc_sh.at[pl.ds(s * ROWS_PER_TILE, ROWS_PER_TILE)],
        z_out.at[c, pl.ds(s * ROWS_PER_TILE, ROWS_PER_TILE)],
    )


def _tc1_body(x_ref, w_ref, degp_ref, y_ref, dinv_ref):
    deg = degp_ref[0] + degp_ref[1] + 1.0
    dinv = lax.rsqrt(jnp.maximum(deg, 1.0))
    h = jnp.dot(x_ref[...], w_ref[...], preferred_element_type=jnp.float32)
    pad = jnp.zeros((N, DP - D), jnp.float32)
    y = jnp.concatenate([h * dinv, pad], axis=1)
    y_ref[...] = jnp.concatenate(
        [y, jnp.zeros((N_PAD - N, DP), jnp.float32)], axis=0
    )
    dinv_ref[...] = dinv


def _tc2_body(zp_ref, y1_ref, dinv_ref, w2_ref, h_ref, y2_ref):
    z8 = zp_ref[0] + zp_ref[1] + y1_ref[:N]
    g8 = jnp.tanh(z8 * dinv_ref[...])
    g = g8[:, :D]
    h_ref[...] = g
    pad = jnp.zeros((N, DP - D), jnp.float32)
    y2 = jnp.dot(g, w2_ref[...], preferred_element_type=jnp.float32) * dinv_ref[...]
    y2_ref[...] = jnp.concatenate(
        [jnp.concatenate([y2, pad], axis=1),
         jnp.zeros((N_PAD - N, DP), jnp.float32)], axis=0
    )


def _tc3_body(zp_ref, y2_ref, dinv_ref, out_ref):
    z8 = zp_ref[0] + zp_ref[1] + y2_ref[:N]
    out_ref[...] = z8[:, :D] * dinv_ref[...]


def kernel(x, edge_index, W1, W2):
    src = edge_index[0].astype(jnp.int32)
    dst = edge_index[1].astype(jnp.int32)
    pad = E_PAD - E
    src3 = jnp.concatenate([src, jnp.zeros((pad,), jnp.int32)]).reshape(
        NW, CPW, CHUNK
    )
    dst3 = jnp.concatenate([dst, jnp.full((pad,), N, jnp.int32)]).reshape(
        NW, CPW, CHUNK
    )
    ones8 = jnp.ones((CHUNK, DP), jnp.float32)
    zer8 = jnp.zeros((N_PAD, DP), jnp.float32)

    degp = _deg_sc(dst3, ones8, zer8)
    deg2 = degp[:, :N, 0:1]

    y1, dinv = pl.pallas_call(
        _tc1_body,
        out_shape=[
            jax.ShapeDtypeStruct((N_PAD, DP), jnp.float32),
            jax.ShapeDtypeStruct((N, 1), jnp.float32),
        ],
    )(x, W1, deg2)

    z1p = _msg_sc(src3, dst3, y1, zer8)

    h, y2 = pl.pallas_call(
        _tc2_body,
        out_shape=[
            jax.ShapeDtypeStruct((N, D), jnp.float32),
            jax.ShapeDtypeStruct((N_PAD, DP), jnp.float32),
        ],
    )(z1p[:, :N], y1, dinv, W2)

    z2p = _msg_sc(src3, dst3, y2, zer8)

    out = pl.pallas_call(
        _tc3_body,
        out_shape=jax.ShapeDtypeStruct((N, D), jnp.float32),
    )(z2p[:, :N], y2, dinv)

    return (out, h)


# speedup vs baseline: 62.1632x; 1.1654x over previous
import functools

import jax
import jax.numpy as jnp
from jax import lax
from jax.experimental import pallas as pl
from jax.experimental.pallas import tpu as pltpu
from jax.experimental.pallas import tpu_sc as plsc

N = 10000
E = 320000
D_IN = 128
D = 4
DP = 8

NC, NS = 2, 16
NW = NC * NS
CHUNK = 125
NBUF = 4
CPW = 80
N_PAD = 10112
ROWS_PER_TILE = N_PAD // NS

_mesh = plsc.VectorSubcoreMesh(core_axis_name="c", subcore_axis_name="s")
_sc_params = pltpu.CompilerParams(use_tc_tiling_on_sc=False)


@functools.partial(
    pl.kernel,
    out_type=jax.ShapeDtypeStruct((NC, N_PAD, DP), jnp.float32),
    mesh=_mesh,
    scratch_types=[
        pltpu.VMEM((CPW, CHUNK), jnp.int32),
        pltpu.VMEM((CHUNK, DP), jnp.float32),
        pltpu.VMEM_SHARED((N_PAD, DP), jnp.float32),
    ],
    compiler_params=_sc_params,
)
def _deg_sc(dst3, ones_hbm, zer_hbm, deg_out, didx, ones_v, acc_sh):
    c = lax.axis_index("c")
    s = lax.axis_index("s")
    wid = c * NS + s
    pltpu.sync_copy(dst3.at[wid], didx)
    pltpu.sync_copy(ones_hbm, ones_v)

    @pl.when(s == 0)
    def _():
        pltpu.sync_copy(zer_hbm, acc_sh)

    plsc.subcore_barrier()

    def body(j, carry):
        pltpu.sync_copy(ones_v, acc_sh.at[didx.at[j]], add=True)
        return carry

    lax.fori_loop(0, CPW, body, 0)
    plsc.subcore_barrier()
    pltpu.sync_copy(
        acc_sh.at[pl.ds(s * ROWS_PER_TILE, ROWS_PER_TILE)],
        deg_out.at[c, pl.ds(s * ROWS_PER_TILE, ROWS_PER_TILE)],
    )


@functools.partial(
    pl.kernel,
    out_type=jax.ShapeDtypeStruct((NC, N_PAD, DP), jnp.float32),
    mesh=_mesh,
    scratch_types=[
        pltpu.VMEM((CPW, CHUNK), jnp.int32),
        pltpu.VMEM((CPW, CHUNK), jnp.int32),
        pltpu.VMEM((NBUF, CHUNK, DP), jnp.float32),
        pltpu.VMEM_SHARED((N_PAD, DP), jnp.float32),
        pltpu.VMEM_SHARED((N_PAD, DP), jnp.float32),
        pltpu.SemaphoreType.DMA((NBUF,)),
    ],
    compiler_params=_sc_params,
)
def _msg_sc(src3, dst3, y_hbm, zer_hbm, z_out, sidx, didx, msgs, acc_sh,
            y_sh, sems):
    c = lax.axis_index("c")
    s = lax.axis_index("s")
    wid = c * NS + s
    pltpu.sync_copy(src3.at[wid], sidx)
    pltpu.sync_copy(dst3.at[wid], didx)

    pltpu.sync_copy(
        y_hbm.at[pl.ds(s * ROWS_PER_TILE, ROWS_PER_TILE)],
        y_sh.at[pl.ds(s * ROWS_PER_TILE, ROWS_PER_TILE)],
    )

    @pl.when(s == 0)
    def _():
        pltpu.sync_copy(zer_hbm, acc_sh)

    plsc.subcore_barrier()

    for b in range(NBUF):
        pltpu.async_copy(y_sh.at[sidx.at[b]], msgs.at[b], sems.at[b])

    def body(jo, carry):
        for b in range(NBUF):
            j = jo * NBUF + b
            pltpu.make_async_copy(y_sh.at[sidx.at[j]], msgs.at[b],
                                  sems.at[b]).wait()
            pltpu.sync_copy(msgs.at[b], acc_sh.at[didx.at[j]], add=True)
            jn = j + NBUF

            @pl.when(jn < CPW)
            def _():
                pltpu.async_copy(y_sh.at[sidx.at[jn]], msgs.at[b],
                                 sems.at[b])
        return carry

    lax.fori_loop(0, CPW // NBUF, body, 0)
    plsc.subcore_barrier()
    pltpu.sync_copy(
        acc_sh.at[pl.ds(s * ROWS_PER_TILE, ROWS_PER_TILE)],
        z_out.at[c, pl.ds(s * ROWS_PER_TILE, ROWS_PER_TILE)],
    )


def _tc1_body(x_ref, w_ref, degp_ref, y_ref, dinv_ref):
    deg = degp_ref[0, :N, 0:1] + degp_ref[1, :N, 0:1] + 1.0
    dinv = lax.rsqrt(jnp.maximum(deg, 1.0))
    h = jnp.dot(x_ref[...], w_ref[...], preferred_element_type=jnp.float32)
    y = jnp.concatenate([h * dinv, jnp.zeros((N, DP - D), jnp.float32)],
                        axis=1)
    y_ref[...] = jnp.concatenate(
        [y, jnp.zeros((N_PAD - N, DP), jnp.float32)], axis=0
    )
    dinv_ref[...] = dinv


def _tc2_body(zp_ref, y1_ref, dinv_ref, w2_ref, h_ref, y2_ref):
    z8 = zp_ref[0, :N] + zp_ref[1, :N] + y1_ref[:N]
    g8 = jnp.tanh(z8 * dinv_ref[...])
    g = g8[:, :D]
    h_ref[...] = g
    y2 = (jnp.dot(g, w2_ref[...], preferred_element_type=jnp.float32)
          * dinv_ref[...])
    y2 = jnp.concatenate([y2, jnp.zeros((N, DP - D), jnp.float32)], axis=1)
    y2_ref[...] = jnp.concatenate(
        [y2, jnp.zeros((N_PAD - N, DP), jnp.float32)], axis=0
    )


def _tc3_body(zp_ref, y2_ref, dinv_ref, out_ref):
    z8 = zp_ref[0, :N] + zp_ref[1, :N] + y2_ref[:N]
    out_ref[...] = z8[:, :D] * dinv_ref[...]


def kernel(x, edge_index, W1, W2):
    ei = edge_index.astype(jnp.int32)
    src3 = ei[0].reshape(NW, CPW, CHUNK)
    dst3 = ei[1].reshape(NW, CPW, CHUNK)
    ones8 = jnp.ones((CHUNK, DP), jnp.float32)
    zer8 = jnp.zeros((N_PAD, DP), jnp.float32)

    degp = _deg_sc(dst3, ones8, zer8)

    y1, dinv = pl.pallas_call(
        _tc1_body,
        out_shape=[
            jax.ShapeDtypeStruct((N_PAD, DP), jnp.float32),
            jax.ShapeDtypeStruct((N, 1), jnp.float32),
        ],
    )(x, W1, degp)

    z1p = _msg_sc(src3, dst3, y1, zer8)

    h, y2 = pl.pallas_call(
        _tc2_body,
        out_shape=[
            jax.ShapeDtypeStruct((N, D), jnp.float32),
            jax.ShapeDtypeStruct((N_PAD, DP), jnp.float32),
        ],
    )(z1p, y1, dinv, W2)

    z2p = _msg_sc(src3, dst3, y2, zer8)

    out = pl.pallas_call(
        _tc3_body,
        out_shape=jax.ShapeDtypeStruct((N, D), jnp.float32),
    )(z2p, y2, dinv)

    return (out, h)
